# baseline (speedup 1.0000x reference)
import functools

import jax
import jax.numpy as jnp
from jax import lax
from jax.experimental import pallas as pl
from jax.experimental.pallas import tpu as pltpu
from jax.experimental.pallas import tpu_sc as plsc

_B = 16384
_MF_D = 32
_MLP_D = 64
_MF_W = _MF_D // 2
_MLP_W = _MLP_D // 2
_NC, _NS = 2, 16
_NW = _NC * _NS
_BPW = _B // _NW
_CHUNK = 128
_NCHUNK = _BPW // _CHUNK

_TC_BLK = 2048


def _sc_gather_kernel(uidx_hbm, iidx_hbm, mfu_hbm, mfi_hbm, mlu_hbm, mli_hbm,
                      out_mfu, out_mfi, out_mlu, out_mli,
                      uidx_v, iidx_v, mfu_v, mfi_v, mlu_v, mli_v, sem):
    wid = lax.axis_index("s") * _NC + lax.axis_index("c")
    base = wid * _BPW
    pltpu.sync_copy(uidx_hbm.at[wid], uidx_v)
    pltpu.sync_copy(iidx_hbm.at[wid], iidx_v)
    copies = []
    for j in range(_NCHUNK):
        off = j * _CHUNK
        copies.append(pltpu.async_copy(
            mfu_hbm.at[uidx_v.at[j]], mfu_v.at[pl.ds(off, _CHUNK)], sem))
        copies.append(pltpu.async_copy(
            mfi_hbm.at[iidx_v.at[j]], mfi_v.at[pl.ds(off, _CHUNK)], sem))
        copies.append(pltpu.async_copy(
            mlu_hbm.at[uidx_v.at[j]], mlu_v.at[pl.ds(off, _CHUNK)], sem))
        copies.append(pltpu.async_copy(
            mli_hbm.at[iidx_v.at[j]], mli_v.at[pl.ds(off, _CHUNK)], sem))
    for cp in copies:
        cp.wait()
    pltpu.sync_copy(mfu_v, out_mfu.at[pl.ds(base, _BPW)])
    pltpu.sync_copy(mfi_v, out_mfi.at[pl.ds(base, _BPW)])
    pltpu.sync_copy(mlu_v, out_mlu.at[pl.ds(base, _BPW)])
    pltpu.sync_copy(mli_v, out_mli.at[pl.ds(base, _BPW)])


_sc_gather = functools.partial(
    pl.kernel,
    mesh=plsc.VectorSubcoreMesh(core_axis_name="c", subcore_axis_name="s"),
    out_type=[
        jax.ShapeDtypeStruct((_B, _MF_W), jnp.int32),
        jax.ShapeDtypeStruct((_B, _MF_W), jnp.int32),
        jax.ShapeDtypeStruct((_B, _MLP_W), jnp.int32),
        jax.ShapeDtypeStruct((_B, _MLP_W), jnp.int32),
    ],
    scratch_types=[
        pltpu.VMEM((_NCHUNK, _CHUNK), jnp.int32),
        pltpu.VMEM((_NCHUNK, _CHUNK), jnp.int32),
        pltpu.VMEM((_BPW, _MF_W), jnp.int32),
        pltpu.VMEM((_BPW, _MF_W), jnp.int32),
        pltpu.VMEM((_BPW, _MLP_W), jnp.int32),
        pltpu.VMEM((_BPW, _MLP_W), jnp.int32),
        pltpu.SemaphoreType.DMA,
    ],
    compiler_params=pltpu.CompilerParams(use_tc_tiling_on_sc=False),
)(_sc_gather_kernel)


def _tc_mlp_kernel(mfu, mfi, mlu, mli, w1u, w1i, b1, w2, b2, wfm, wfh, bf, out):
    f32 = lambda x: x[...].astype(jnp.float32)
    h = jnp.dot(f32(mlu), w1u[...], preferred_element_type=jnp.float32)
    h = h + jnp.dot(f32(mli), w1i[...], preferred_element_type=jnp.float32)
    h = jnp.maximum(h + b1[...], 0.0)
    h = jnp.dot(h, w2[...], preferred_element_type=jnp.float32) + b2[...]
    h = jnp.maximum(h, 0.0)
    mf = f32(mfu) * f32(mfi)
    out[...] = (jnp.dot(mf, wfm[...], preferred_element_type=jnp.float32)
                + jnp.dot(h, wfh[...], preferred_element_type=jnp.float32)
                + bf[...])


def _tc_mlp(mfu, mfi, mlu, mli, w1u, w1i, b1, w2, b2, wfm, wfh, bf):
    grid = _B // _TC_BLK
    row_spec = lambda d: pl.BlockSpec((_TC_BLK, d), lambda i: (i, 0))
    full = lambda a: pl.BlockSpec(a.shape, lambda i: (0,) * a.ndim)
    return pl.pallas_call(
        _tc_mlp_kernel,
        grid=(grid,),
        in_specs=[
            row_spec(_MF_D), row_spec(_MF_D), row_spec(_MLP_D), row_spec(_MLP_D),
            full(w1u), full(w1i), full(b1), full(w2), full(b2),
            full(wfm), full(wfh), full(bf),
        ],
        out_specs=pl.BlockSpec((_TC_BLK, 1), lambda i: (i, 0)),
        out_shape=jax.ShapeDtypeStruct((_B, 1), jnp.float32),
    )(mfu, mfi, mlu, mli, w1u, w1i, b1, w2, b2, wfm, wfh, bf)


def _pack_bf16(table):
    n, d = table.shape
    t16 = table.astype(jnp.bfloat16).reshape(n, d // 2, 2)
    return lax.bitcast_convert_type(t16, jnp.int32)


def _unpack_bf16(packed):
    n, w = packed.shape
    return lax.bitcast_convert_type(packed, jnp.bfloat16).reshape(n, 2 * w)


def kernel(user_input, item_input, mf_user_emb, mf_item_emb,
           mlp_user_emb, mlp_item_emb, W1, b1, W2, b2, Wf, bf):
    uidx = user_input.astype(jnp.int32).reshape(_NW, _NCHUNK, _CHUNK)
    iidx = item_input.astype(jnp.int32).reshape(_NW, _NCHUNK, _CHUNK)
    mfu_p, mfi_p, mlu_p, mli_p = _sc_gather(
        uidx, iidx,
        _pack_bf16(mf_user_emb), _pack_bf16(mf_item_emb),
        _pack_bf16(mlp_user_emb), _pack_bf16(mlp_item_emb))
    w1u = W1[:_MLP_D]
    w1i = W1[_MLP_D:]
    wfm = Wf[:_MF_D]
    wfh = Wf[_MF_D:]
    out = _tc_mlp(_unpack_bf16(mfu_p), _unpack_bf16(mfi_p),
                  _unpack_bf16(mlu_p), _unpack_bf16(mli_p),
                  w1u, w1i, b1.reshape(1, -1), W2, b2.reshape(1, -1),
                  wfm, wfh, bf.reshape(1, 1))
    return out
